# baseline (speedup 1.0000x reference)
import jax
import jax.numpy as jnp
import numpy as np
from jax.experimental import pallas as pl
from jax.experimental.pallas import tpu as pltpu

_T = 150
_R = 36
_N = _T * _R
_D = 128
_DS = 64
_F = 6
_C = _T // _F
_FR = _F * _R
_WIN = (_F + 2) * _R


def _masks():
    row = np.arange(_FR)[:, None] // _R
    col = np.arange(_WIN)[None, :] // _R
    m = np.stack([np.abs(col - row - v) <= 1 for v in (0, 1, 2)])
    return (1.0 - m.astype(np.float32)) * np.float32(-1e30)


_MASKS = _masks()


def _src(l, x_ref, s0, s1, start, size):
    return jnp.where(
        l == 0, x_ref[pl.ds(start, size), :],
        jnp.where(l == 1, s0[pl.ds(start, size), :], s1[pl.ds(start, size), :]))


def _body(x_ref, wq_ref, wk_ref, w_ref, b_ref, m_ref, out_ref, s0, s1, sbuf):
    l = pl.program_id(0)
    i = pl.program_id(1)

    ca = jnp.minimum(i, _C - 1)
    base_a = jnp.clip(ca * _FR - _R, 0, _N - _WIN)
    xw_a = _src(l, x_ref, s0, s1, base_a, _WIN)
    xq = _src(l, x_ref, s0, s1, ca * _FR, _FR)
    q = jnp.dot(xq, wq_ref[0], preferred_element_type=jnp.float32)
    k = jnp.dot(xw_a, wk_ref[0], preferred_element_type=jnp.float32)
    s = jax.lax.dot_general(q, k, (((1,), (1,)), ((), ())),
                            preferred_element_type=jnp.float32)
    sbuf[pl.ds(jax.lax.rem(i, 2) * _FR, _FR), :] = s + m_ref[0]

    j = jnp.maximum(i - 1, 0)
    base_b = jnp.clip(j * _FR - _R, 0, _N - _WIN)
    sj = sbuf[pl.ds(jax.lax.rem(i + 1, 2) * _FR, _FR), :]
    e = jnp.exp(sj - jnp.max(sj, axis=1, keepdims=True))
    p = e / jnp.sum(e, axis=1, keepdims=True)
    xw_b = _src(l, x_ref, s0, s1, base_b, _WIN)
    agg = jnp.dot(p, xw_b, preferred_element_type=jnp.float32)
    h = jnp.dot(agg, w_ref[0], preferred_element_type=jnp.float32) + b_ref[0]
    h = jnp.where(l < 2, jnp.maximum(h, 0.0), h)

    @pl.when(jnp.logical_and(l == 0, i > 0))
    def _w0():
        s0[pl.ds(j * _FR, _FR), :] = h

    @pl.when(jnp.logical_and(l == 1, i > 0))
    def _w1():
        s1[pl.ds(j * _FR, _FR), :] = h

    @pl.when(jnp.logical_and(l == 2, i > 0))
    def _w2():
        out_ref[:, :] = h


def kernel(x, Wq1, Wk1, W1, b1, Wq2, Wk2, W2, b2, Wq3, Wk3, W3, b3,
           num_frames, num_regions):
    wq = jnp.stack([Wq1, Wq2, Wq3])
    wk = jnp.stack([Wk1, Wk2, Wk3])
    w = jnp.stack([W1, W2, W3])
    b = jnp.stack([b1, b2, b3]).reshape(3, 1, _D)

    lmap = lambda l, i: (l, 0, 0)
    full = lambda l, i: (0, 0)
    mmap = lambda l, i: (jnp.where(i == 0, 0, jnp.where(i == _C - 1, 2, 1)),
                         0, 0)

    return pl.pallas_call(
        _body,
        grid=(3, _C + 1),
        in_specs=[
            pl.BlockSpec((_N, _D), full),
            pl.BlockSpec((1, _D, _DS), lmap),
            pl.BlockSpec((1, _D, _DS), lmap),
            pl.BlockSpec((1, _D, _D), lmap),
            pl.BlockSpec((1, 1, _D), lmap),
            pl.BlockSpec((1, _FR, _WIN), mmap),
        ],
        out_specs=pl.BlockSpec(
            (_FR, _D),
            lambda l, i: (jnp.where(l == 2, jnp.maximum(i - 1, 0), 0), 0)),
        out_shape=jax.ShapeDtypeStruct((_N, _D), jnp.float32),
        scratch_shapes=[
            pltpu.VMEM((_N, _D), jnp.float32),
            pltpu.VMEM((_N, _D), jnp.float32),
            pltpu.VMEM((2 * _FR, _WIN), jnp.float32),
        ],
    )(x, wq, wk, w, b, jnp.asarray(_MASKS))


# speedup vs baseline: 1.2376x; 1.2376x over previous
import jax
import jax.numpy as jnp
import numpy as np
from jax.experimental import pallas as pl
from jax.experimental.pallas import tpu as pltpu

_T = 150
_R = 36
_N = _T * _R
_D = 128
_DS = 64
_F = 5
_SUB = 2
_C = _T // (_F * _SUB)
_FR = _F * _R
_WIN = (_F + 2) * _R


def _masks():
    row = np.arange(_FR)[:, None] // _R
    col = np.arange(_WIN)[None, :] // _R
    m = np.stack([np.abs(col - row - v) <= 1 for v in (0, 1, 2)])
    return (1.0 - m.astype(np.float32)) * np.float32(-1e30)


_MASKS = _masks()


def _src(l, x_ref, s0, s1, start, size):
    return jnp.where(
        l == 0, x_ref[pl.ds(start, size), :],
        jnp.where(l == 1, s0[pl.ds(start, size), :], s1[pl.ds(start, size), :]))


def _body(x_ref, wq_ref, wk_ref, w_ref, b_ref, m0_ref, m1_ref, out_ref,
          s0, s1):
    l = pl.program_id(0)
    i = pl.program_id(1)

    xws, ss = [], []
    for sub, m_ref in ((0, m0_ref), (1, m1_ref)):
        qb = (i * _SUB + sub) * _FR
        base = jnp.clip(qb - _R, 0, _N - _WIN)
        xw = _src(l, x_ref, s0, s1, base, _WIN)
        xq = _src(l, x_ref, s0, s1, qb, _FR)
        q = jnp.dot(xq, wq_ref[0], preferred_element_type=jnp.float32)
        k = jnp.dot(xw, wk_ref[0], preferred_element_type=jnp.float32)
        s = jax.lax.dot_general(q, k, (((1,), (1,)), ((), ())),
                                preferred_element_type=jnp.float32)
        xws.append(xw)
        ss.append(s + m_ref[0])

    for sub in (0, 1):
        s = ss[sub]
        e = jnp.exp(s - jnp.max(s, axis=1, keepdims=True))
        p = e / jnp.sum(e, axis=1, keepdims=True)
        agg = jnp.dot(p, xws[sub], preferred_element_type=jnp.float32)
        h = jnp.dot(agg, w_ref[0], preferred_element_type=jnp.float32) + b_ref[0]
        h = jnp.where(l < 2, jnp.maximum(h, 0.0), h)
        qb = (i * _SUB + sub) * _FR

        @pl.when(l == 0)
        def _w0():
            s0[pl.ds(qb, _FR), :] = h

        @pl.when(l == 1)
        def _w1():
            s1[pl.ds(qb, _FR), :] = h

        @pl.when(l == 2)
        def _w2():
            out_ref[pl.ds(sub * _FR, _FR), :] = h


def kernel(x, Wq1, Wk1, W1, b1, Wq2, Wk2, W2, b2, Wq3, Wk3, W3, b3,
           num_frames, num_regions):
    wq = jnp.stack([Wq1, Wq2, Wq3])
    wk = jnp.stack([Wk1, Wk2, Wk3])
    w = jnp.stack([W1, W2, W3])
    b = jnp.stack([b1, b2, b3]).reshape(3, 1, _D)

    lmap = lambda l, i: (l, 0, 0)
    full = lambda l, i: (0, 0)
    m0map = lambda l, i: (jnp.where(i == 0, 0, 1), 0, 0)
    m1map = lambda l, i: (jnp.where(i == _C - 1, 2, 1), 0, 0)

    masks = jnp.asarray(_MASKS)
    return pl.pallas_call(
        _body,
        grid=(3, _C),
        in_specs=[
            pl.BlockSpec((_N, _D), full),
            pl.BlockSpec((1, _D, _DS), lmap),
            pl.BlockSpec((1, _D, _DS), lmap),
            pl.BlockSpec((1, _D, _D), lmap),
            pl.BlockSpec((1, 1, _D), lmap),
            pl.BlockSpec((1, _FR, _WIN), m0map),
            pl.BlockSpec((1, _FR, _WIN), m1map),
        ],
        out_specs=pl.BlockSpec(
            (_SUB * _FR, _D),
            lambda l, i: (jnp.where(l == 2, i, 0), 0)),
        out_shape=jax.ShapeDtypeStruct((_N, _D), jnp.float32),
        scratch_shapes=[
            pltpu.VMEM((_N, _D), jnp.float32),
            pltpu.VMEM((_N, _D), jnp.float32),
        ],
    )(x, wq, wk, w, b, masks, masks)


# speedup vs baseline: 1.2478x; 1.0083x over previous
import jax
import jax.numpy as jnp
import numpy as np
from jax.experimental import pallas as pl
from jax.experimental.pallas import tpu as pltpu

_T = 150
_R = 36
_N = _T * _R
_D = 128
_DS = 64
_F = 5
_SUB = 6
_C = _T // (_F * _SUB)
_FR = _F * _R
_WIN = (_F + 2) * _R


def _masks():
    row = np.arange(_FR)[:, None] // _R
    col = np.arange(_WIN)[None, :] // _R
    m = np.stack([np.abs(col - row - v) <= 1 for v in (0, 1, 2)])
    return (1.0 - m.astype(np.float32)) * np.float32(-1e30)


_MASKS = _masks()


def _src(l, x_ref, s0, s1, start, size):
    return jnp.where(
        l == 0, x_ref[pl.ds(start, size), :],
        jnp.where(l == 1, s0[pl.ds(start, size), :], s1[pl.ds(start, size), :]))


def _body(x_ref, wq_ref, wk_ref, w_ref, b_ref, *rest):
    m_refs = rest[:_SUB]
    out_ref = rest[_SUB]
    s0, s1 = rest[_SUB + 1], rest[_SUB + 2]
    l = pl.program_id(0)
    i = pl.program_id(1)

    xws, ss = [], []
    for sub, m_ref in enumerate(m_refs):
        qb = (i * _SUB + sub) * _FR
        base = jnp.clip(qb - _R, 0, _N - _WIN)
        xw = _src(l, x_ref, s0, s1, base, _WIN)
        xq = _src(l, x_ref, s0, s1, qb, _FR)
        q = jnp.dot(xq, wq_ref[0], preferred_element_type=jnp.float32)
        k = jnp.dot(xw, wk_ref[0], preferred_element_type=jnp.float32)
        s = jax.lax.dot_general(q, k, (((1,), (1,)), ((), ())),
                                preferred_element_type=jnp.float32)
        xws.append(xw)
        ss.append(s + m_ref[0])

    for sub in range(_SUB):
        s = ss[sub]
        e = jnp.exp(s - jnp.max(s, axis=1, keepdims=True))
        p = e / jnp.sum(e, axis=1, keepdims=True)
        agg = jnp.dot(p, xws[sub], preferred_element_type=jnp.float32)
        h = jnp.dot(agg, w_ref[0], preferred_element_type=jnp.float32) + b_ref[0]
        h = jnp.where(l < 2, jnp.maximum(h, 0.0), h)
        qb = (i * _SUB + sub) * _FR

        @pl.when(l == 0)
        def _w0():
            s0[pl.ds(qb, _FR), :] = h

        @pl.when(l == 1)
        def _w1():
            s1[pl.ds(qb, _FR), :] = h

        @pl.when(l == 2)
        def _w2():
            out_ref[pl.ds(sub * _FR, _FR), :] = h


def kernel(x, Wq1, Wk1, W1, b1, Wq2, Wk2, W2, b2, Wq3, Wk3, W3, b3,
           num_frames, num_regions):
    wq = jnp.stack([Wq1, Wq2, Wq3])
    wk = jnp.stack([Wk1, Wk2, Wk3])
    w = jnp.stack([W1, W2, W3])
    b = jnp.stack([b1, b2, b3]).reshape(3, 1, _D)

    lmap = lambda l, i: (l, 0, 0)
    full = lambda l, i: (0, 0)
    def _mmap(sub):
        if sub == 0:
            return lambda l, i: (jnp.where(i == 0, 0, 1), 0, 0)
        if sub == _SUB - 1:
            return lambda l, i: (jnp.where(i == _C - 1, 2, 1), 0, 0)
        return lambda l, i: (1, 0, 0)

    masks = jnp.asarray(_MASKS)
    return pl.pallas_call(
        _body,
        grid=(3, _C),
        in_specs=[
            pl.BlockSpec((_N, _D), full),
            pl.BlockSpec((1, _D, _DS), lmap),
            pl.BlockSpec((1, _D, _DS), lmap),
            pl.BlockSpec((1, _D, _D), lmap),
            pl.BlockSpec((1, 1, _D), lmap),
            *[pl.BlockSpec((1, _FR, _WIN), _mmap(sub)) for sub in range(_SUB)],
        ],
        out_specs=pl.BlockSpec(
            (_SUB * _FR, _D),
            lambda l, i: (jnp.where(l == 2, i, 0), 0)),
        out_shape=jax.ShapeDtypeStruct((_N, _D), jnp.float32),
        scratch_shapes=[
            pltpu.VMEM((_N, _D), jnp.float32),
            pltpu.VMEM((_N, _D), jnp.float32),
        ],
    )(x, wq, wk, w, b, *([masks] * _SUB))
